# Initial kernel scaffold; baseline (speedup 1.0000x reference)
import jax
import jax.numpy as jnp
from jax.experimental import pallas as pl


def kernel(x, edge_index, edge_attr, batch, params):
    raise NotImplementedError("write your pallas kernel here")


# speedup vs baseline: 2.6710x; 2.6710x over previous
import functools

import jax
import jax.numpy as jnp
from jax import lax
from jax.experimental import pallas as pl
from jax.experimental.pallas import tpu as pltpu
from jax.experimental.pallas import tpu_sc as plsc

N = 10000
E = 160000
NUM_NBR = 16
AFL = 64
NFL = 128
H = 128
NGRAPH = 64

BLK_E = 3200
NBLK_E = E // BLK_E
BLK_N = 2000
NBLK_N = N // BLK_N

TWO_E = 2 * E
_NW = 32
_BPW = TWO_E // _NW
_CH = 80
_NCHUNK = _BPW // _CH

_ATOM_DIMS = [100, 18, 7, 12, 10, 10, 10, 10, 10]
_ATOM_OFF = [0, 100, 118, 125, 137, 147, 157, 167, 177]
_ATOM_TOT_PAD = 192


def _softplus(x):
    return jnp.maximum(x, 0.0) + jnp.log(1.0 + jnp.exp(-jnp.abs(x)))


def _sigmoid(x):
    return 1.0 / (1.0 + jnp.exp(-x))


_GW = 128


def _gather_edges(h, idx_all):
    mesh = plsc.VectorSubcoreMesh(core_axis_name="c", subcore_axis_name="s")

    @functools.partial(
        pl.kernel,
        mesh=mesh,
        out_type=jax.ShapeDtypeStruct((TWO_E, _GW), jnp.float32),
        scratch_types=[
            pltpu.VMEM((_BPW,), jnp.int32),
            pltpu.VMEM((_CH, _GW), jnp.float32),
            pltpu.SemaphoreType.DMA,
        ],
    )
    def k(h_hbm, idx_hbm, out_hbm, idx_v, rows_v, sem):
        wid = lax.axis_index("s") * 2 + lax.axis_index("c")
        base = wid * _BPW
        pltpu.sync_copy(idx_hbm.at[pl.ds(base, _BPW)], idx_v)

        def body(i, carry):
            off = pl.multiple_of(i * _CH, 8)
            pltpu.async_copy(
                h_hbm.at[idx_v.at[pl.ds(off, _CH)]], rows_v, sem
            ).wait()
            pltpu.sync_copy(rows_v, out_hbm.at[pl.ds(base + off, _CH)])
            return carry

        lax.fori_loop(0, _NCHUNK, body, 0)

    return k(h, idx_all)


def _atom_embed(x3, atom_tab):
    def body(x_r, tab_r, o_r):
        xb = x_r[0]
        mh = jnp.zeros((BLK_N, _ATOM_TOT_PAD), jnp.float32)
        cols = jax.lax.broadcasted_iota(jnp.int32, (BLK_N, _ATOM_TOT_PAD), 1)
        for i in range(9):
            tgt = xb[:, i] + _ATOM_OFF[i]
            mh = mh + (cols == tgt[:, None]).astype(jnp.float32)
        o_r[...] = jnp.dot(mh, tab_r[...], preferred_element_type=jnp.float32)

    return pl.pallas_call(
        body,
        grid=(NBLK_N,),
        in_specs=[
            pl.BlockSpec((1, BLK_N, 9), lambda i: (i, 0, 0)),
            pl.BlockSpec((_ATOM_TOT_PAD, AFL), lambda i: (0, 0)),
        ],
        out_specs=pl.BlockSpec((BLK_N, AFL), lambda i: (i, 0)),
        out_shape=jax.ShapeDtypeStruct((N, AFL), jnp.float32),
    )(x3, atom_tab)


def _bond_embed(attr3, bond_tab):
    def body(a_r, tab_r, o_r):
        a = a_r[0, 0]
        cols = jax.lax.broadcasted_iota(jnp.int32, (BLK_E, 64), 1)
        oh = (cols == a[:, None]).astype(jnp.float32)
        o_r[...] = jnp.dot(oh, tab_r[...], preferred_element_type=jnp.float32)

    return pl.pallas_call(
        body,
        grid=(NBLK_E,),
        in_specs=[
            pl.BlockSpec((1, 1, BLK_E), lambda i: (i, 0, 0)),
            pl.BlockSpec((64, NFL), lambda i: (0, 0)),
        ],
        out_specs=pl.BlockSpec((BLK_E, NFL), lambda i: (i, 0)),
        out_shape=jax.ShapeDtypeStruct((E, NFL), jnp.float32),
    )(attr3, bond_tab)


def _edge_t(anbr_r, ainit_r, ea_r, W_r, b_r):
    return (
        jnp.dot(anbr_r[:, 0:AFL], W_r[0:AFL, :],
                preferred_element_type=jnp.float32)
        + jnp.dot(ainit_r[:, 0:AFL], W_r[AFL:2 * AFL, :],
                  preferred_element_type=jnp.float32)
        + jnp.dot(ea_r[...], W_r[2 * AFL:, :],
                  preferred_element_type=jnp.float32)
        + b_r[...]
    )


def _conv_stats(gath, ew, Wcat, bcat):
    def body(anbr_r, ainit_r, ea_r, W_r, b_r, acc_r):
        @pl.when(pl.program_id(0) == 0)
        def _init():
            acc_r[...] = jnp.zeros((2, 2 * AFL + NFL), jnp.float32)

        t = _edge_t(anbr_r, ainit_r, ea_r, W_r, b_r)
        s = jnp.sum(t, axis=0)
        ss = jnp.sum(t * t, axis=0)
        acc_r[...] += jnp.concatenate([s[None, :], ss[None, :]], axis=0)

    return pl.pallas_call(
        body,
        grid=(NBLK_E,),
        in_specs=[
            pl.BlockSpec((BLK_E, _GW), lambda i: (i, 0)),
            pl.BlockSpec((BLK_E, _GW), lambda i: (i + NBLK_E, 0)),
            pl.BlockSpec((BLK_E, NFL), lambda i: (i, 0)),
            pl.BlockSpec((2 * AFL + NFL, 2 * AFL + NFL), lambda i: (0, 0)),
            pl.BlockSpec((1, 2 * AFL + NFL), lambda i: (0, 0)),
        ],
        out_specs=pl.BlockSpec((2, 2 * AFL + NFL), lambda i: (0, 0)),
        out_shape=jax.ShapeDtypeStruct((2, 2 * AFL + NFL), jnp.float32),
    )(gath, gath, ew, Wcat, bcat)


def _conv_apply(gath, ew, Wcat, bcat, acc, gcat, betacat):
    def body(anbr_r, ainit_r, ea_r, W_r, b_r, acc_r, g_r, bt_r, nbr_r, ewo_r):
        t = _edge_t(anbr_r, ainit_r, ea_r, W_r, b_r)
        m = acc_r[0, :] * (1.0 / E)
        var = acc_r[1, :] * (1.0 / E) - m * m
        rstd = jax.lax.rsqrt(var + 1e-5)
        tn = (t - m[None, :]) * (rstd * g_r[0, :])[None, :] + bt_r[0, :][None, :]
        filt = _sigmoid(tn[:, 0:AFL])
        core = _softplus(tn[:, AFL:2 * AFL])
        prod = filt * core
        nbr_r[...] = jnp.sum(
            prod.reshape(BLK_E // NUM_NBR, NUM_NBR, AFL), axis=1)
        ewo_r[...] = _softplus(ea_r[...] + tn[:, 2 * AFL:])

    return pl.pallas_call(
        body,
        grid=(NBLK_E,),
        in_specs=[
            pl.BlockSpec((BLK_E, _GW), lambda i: (i, 0)),
            pl.BlockSpec((BLK_E, _GW), lambda i: (i + NBLK_E, 0)),
            pl.BlockSpec((BLK_E, NFL), lambda i: (i, 0)),
            pl.BlockSpec((2 * AFL + NFL, 2 * AFL + NFL), lambda i: (0, 0)),
            pl.BlockSpec((1, 2 * AFL + NFL), lambda i: (0, 0)),
            pl.BlockSpec((2, 2 * AFL + NFL), lambda i: (0, 0)),
            pl.BlockSpec((1, 2 * AFL + NFL), lambda i: (0, 0)),
            pl.BlockSpec((1, 2 * AFL + NFL), lambda i: (0, 0)),
        ],
        out_specs=(
            pl.BlockSpec((BLK_E // NUM_NBR, AFL), lambda i: (i, 0)),
            pl.BlockSpec((BLK_E, NFL), lambda i: (i, 0)),
        ),
        out_shape=(
            jax.ShapeDtypeStruct((N, AFL), jnp.float32),
            jax.ShapeDtypeStruct((E, NFL), jnp.float32),
        ),
    )(gath, gath, ew, Wcat, bcat, acc, gcat, betacat)


def _node_update(h, nbr, g4, b4):
    def body(h_r, n_r, g_r, b_r, o_r):
        nb = n_r[...]
        m = jnp.sum(nb, axis=0) * (1.0 / N)
        d = nb - m[None, :]
        var = jnp.sum(d * d, axis=0) * (1.0 / N)
        rstd = jax.lax.rsqrt(var + 1e-5)
        o_r[...] = _softplus(h_r[...] + d * (rstd * g_r[0, :])[None, :]
                             + b_r[0, :][None, :])

    return pl.pallas_call(
        body,
        in_specs=[
            pl.BlockSpec((N, AFL), lambda: (0, 0)),
            pl.BlockSpec((N, AFL), lambda: (0, 0)),
            pl.BlockSpec((1, AFL), lambda: (0, 0)),
            pl.BlockSpec((1, AFL), lambda: (0, 0)),
        ],
        out_specs=pl.BlockSpec((N, AFL), lambda: (0, 0)),
        out_shape=jax.ShapeDtypeStruct((N, AFL), jnp.float32),
    )(h, nbr, g4, b4)


def _head_pool(h0, h1, h2, h3, batch3, Wfc, bfc, l1W, l1b):
    def body(h0_r, h1_r, h2_r, h3_r, b_r, Wfc_r, bfc_r, l1W_r, l1b_r,
             s_r, c_r):
        @pl.when(pl.program_id(0) == 0)
        def _init():
            s_r[...] = jnp.zeros((NGRAPH, H), jnp.float32)
            c_r[...] = jnp.zeros((1, NGRAPH), jnp.float32)

        z = (
            jnp.dot(h0_r[...], Wfc_r[0:AFL, :],
                    preferred_element_type=jnp.float32)
            + jnp.dot(h1_r[...], Wfc_r[AFL:2 * AFL, :],
                      preferred_element_type=jnp.float32)
            + jnp.dot(h2_r[...], Wfc_r[2 * AFL:3 * AFL, :],
                      preferred_element_type=jnp.float32)
            + jnp.dot(h3_r[...], Wfc_r[3 * AFL:, :],
                      preferred_element_type=jnp.float32)
            + bfc_r[...]
        )
        z = _softplus(jnp.dot(z, l1W_r[...],
                              preferred_element_type=jnp.float32) + l1b_r[...])
        b = b_r[0, 0]
        rows = jax.lax.broadcasted_iota(jnp.int32, (NGRAPH, BLK_N), 0)
        ohT = (rows == b[None, :]).astype(jnp.float32)
        s_r[...] += jnp.dot(ohT, z, preferred_element_type=jnp.float32)
        c_r[...] += jnp.sum(ohT, axis=1)[None, :]

    return pl.pallas_call(
        body,
        grid=(NBLK_N,),
        in_specs=[
            pl.BlockSpec((BLK_N, AFL), lambda i: (i, 0)),
            pl.BlockSpec((BLK_N, AFL), lambda i: (i, 0)),
            pl.BlockSpec((BLK_N, AFL), lambda i: (i, 0)),
            pl.BlockSpec((BLK_N, AFL), lambda i: (i, 0)),
            pl.BlockSpec((1, 1, BLK_N), lambda i: (i, 0, 0)),
            pl.BlockSpec((4 * AFL, H), lambda i: (0, 0)),
            pl.BlockSpec((1, H), lambda i: (0, 0)),
            pl.BlockSpec((H, H), lambda i: (0, 0)),
            pl.BlockSpec((1, H), lambda i: (0, 0)),
        ],
        out_specs=(
            pl.BlockSpec((NGRAPH, H), lambda i: (0, 0)),
            pl.BlockSpec((1, NGRAPH), lambda i: (0, 0)),
        ),
        out_shape=(
            jax.ShapeDtypeStruct((NGRAPH, H), jnp.float32),
            jax.ShapeDtypeStruct((1, NGRAPH), jnp.float32),
        ),
    )(h0, h1, h2, h3, batch3, Wfc, bfc, l1W, l1b)


def _head_out(s, cnt, l2W, l2b, WoutP, boutP):
    def body(s_r, c_r, l2W_r, l2b_r, Wo_r, bo_r, o_r):
        c = jnp.maximum(c_r[0, :], 1.0)
        mean = s_r[...] * (1.0 / c)[:, None]
        z = _softplus(jnp.dot(mean, l2W_r[...],
                              preferred_element_type=jnp.float32) + l2b_r[...])
        o_r[...] = jnp.dot(z, Wo_r[...],
                           preferred_element_type=jnp.float32) + bo_r[...]

    return pl.pallas_call(
        body,
        in_specs=[
            pl.BlockSpec((NGRAPH, H), lambda: (0, 0)),
            pl.BlockSpec((1, NGRAPH), lambda: (0, 0)),
            pl.BlockSpec((H, H), lambda: (0, 0)),
            pl.BlockSpec((1, H), lambda: (0, 0)),
            pl.BlockSpec((H, H), lambda: (0, 0)),
            pl.BlockSpec((1, H), lambda: (0, 0)),
        ],
        out_specs=pl.BlockSpec((NGRAPH, H), lambda: (0, 0)),
        out_shape=jax.ShapeDtypeStruct((NGRAPH, H), jnp.float32),
    )(s, cnt, l2W, l2b, WoutP, boutP)


def kernel(x, edge_index, edge_attr, batch, params):
    x = x.astype(jnp.int32)
    ei = edge_index.astype(jnp.int32)
    attr = edge_attr.astype(jnp.int32)
    batch = batch.astype(jnp.int32)

    idx_all = jnp.concatenate([ei[1], ei[0]])

    atom_tab = jnp.concatenate(params['atom_emb'], axis=0)
    atom_tab = jnp.pad(atom_tab, ((0, _ATOM_TOT_PAD - atom_tab.shape[0]),
                                  (0, 0)))
    bond_tab = jnp.pad(params['bond_emb'], ((0, 64 - 51), (0, 0)))

    h = _atom_embed(x.reshape(NBLK_N, BLK_N, 9), atom_tab)
    ew = _bond_embed(attr.reshape(NBLK_E, 1, BLK_E), bond_tab)

    hs = [h]
    for p in params['convs']:
        Wcat = jnp.concatenate([p['Wc'], p['Wf'], p['Wb']], axis=1)
        bcat = jnp.concatenate([p['bc'], p['bf'], p['bb']]).reshape(1, -1)
        gcat = jnp.concatenate([p['g1'], p['g2'], p['g3']]).reshape(1, -1)
        btcat = jnp.concatenate([p['b1'], p['b2'], p['b3']]).reshape(1, -1)

        gath = _gather_edges(jnp.pad(h, ((0, 0), (0, _GW - AFL))), idx_all)
        acc = _conv_stats(gath, ew, Wcat, bcat)
        nbr, ew = _conv_apply(gath, ew, Wcat, bcat, acc, gcat, btcat)
        h = _node_update(h, nbr, p['g4'].reshape(1, AFL),
                         p['b4'].reshape(1, AFL))
        hs.append(h)

    s, cnt = _head_pool(hs[0], hs[1], hs[2], hs[3],
                        batch.reshape(NBLK_N, 1, BLK_N),
                        params['W_fc'], params['b_fc'].reshape(1, H),
                        params['l1_W'], params['l1_b'].reshape(1, H))
    WoutP = jnp.pad(params['Wout'], ((0, 0), (0, H - 1)))
    boutP = jnp.broadcast_to(params['bout'].reshape(1, 1), (1, H))
    o = _head_out(s, cnt, params['l2_W'], params['l2_b'].reshape(1, H),
                  WoutP, boutP)
    return o[:, 0:1]


# speedup vs baseline: 3.2046x; 1.1998x over previous
import functools

import jax
import jax.numpy as jnp
from jax import lax
from jax.experimental import pallas as pl
from jax.experimental.pallas import tpu as pltpu
from jax.experimental.pallas import tpu_sc as plsc

N = 10000
E = 160000
NUM_NBR = 16
AFL = 64
NFL = 128
H = 128
NGRAPH = 64

BLK_E = 3200
NBLK_E = E // BLK_E
BLK_N = 2000
NBLK_N = N // BLK_N

TWO_E = 2 * E
_NW = 32
_BPW = TWO_E // _NW
_CH = 80
_NBUF = 5
_NGRP = _BPW // (_CH * _NBUF)

_ATOM_DIMS = [100, 18, 7, 12, 10, 10, 10, 10, 10]
_ATOM_OFF = [0, 100, 118, 125, 137, 147, 157, 167, 177]
_ATOM_TOT_PAD = 192


def _softplus(x):
    return jnp.maximum(x, 0.0) + jnp.log(1.0 + jnp.exp(-jnp.abs(x)))


def _sigmoid(x):
    return 1.0 / (1.0 + jnp.exp(-x))


_GW = 128


def _gather_edges(h, idx_all):
    mesh = plsc.VectorSubcoreMesh(core_axis_name="c", subcore_axis_name="s")

    @functools.partial(
        pl.kernel,
        mesh=mesh,
        out_type=jax.ShapeDtypeStruct((TWO_E, _GW), jnp.float32),
        scratch_types=(
            [pltpu.VMEM((_BPW,), jnp.int32)]
            + [pltpu.VMEM((_CH, _GW), jnp.float32) for _ in range(_NBUF)]
            + [pltpu.SemaphoreType.DMA for _ in range(2 * _NBUF)]
        ),
    )
    def k(h_hbm, idx_hbm, out_hbm, idx_v, *rest):
        bufs = rest[:_NBUF]
        gsems = rest[_NBUF:2 * _NBUF]
        wsems = rest[2 * _NBUF:]
        wid = lax.axis_index("s") * 2 + lax.axis_index("c")
        base = wid * _BPW
        pltpu.sync_copy(idx_hbm.at[pl.ds(base, _BPW)], idx_v)

        def group(j, carry):
            gd = []
            for b in range(_NBUF):
                off = pl.multiple_of((j * _NBUF + b) * _CH, 8)
                gd.append(pltpu.async_copy(
                    h_hbm.at[idx_v.at[pl.ds(off, _CH)]], bufs[b], gsems[b]))
            wd = []
            for b in range(_NBUF):
                off = pl.multiple_of((j * _NBUF + b) * _CH, 8)
                gd[b].wait()
                wd.append(pltpu.async_copy(
                    bufs[b], out_hbm.at[pl.ds(base + off, _CH)], wsems[b]))
            for d in wd:
                d.wait()
            return carry

        lax.fori_loop(0, _NGRP, group, 0)

    return k(h, idx_all)


def _atom_embed(x3, atom_tab):
    def body(x_r, tab_r, o_r):
        xb = x_r[0]
        mh = jnp.zeros((BLK_N, _ATOM_TOT_PAD), jnp.float32)
        cols = jax.lax.broadcasted_iota(jnp.int32, (BLK_N, _ATOM_TOT_PAD), 1)
        for i in range(9):
            tgt = xb[:, i] + _ATOM_OFF[i]
            mh = mh + (cols == tgt[:, None]).astype(jnp.float32)
        o_r[...] = jnp.dot(mh, tab_r[...], preferred_element_type=jnp.float32)

    return pl.pallas_call(
        body,
        grid=(NBLK_N,),
        in_specs=[
            pl.BlockSpec((1, BLK_N, 9), lambda i: (i, 0, 0)),
            pl.BlockSpec((_ATOM_TOT_PAD, AFL), lambda i: (0, 0)),
        ],
        out_specs=pl.BlockSpec((BLK_N, AFL), lambda i: (i, 0)),
        out_shape=jax.ShapeDtypeStruct((N, AFL), jnp.float32),
    )(x3, atom_tab)


def _bond_embed(attr3, bond_tab):
    def body(a_r, tab_r, o_r):
        a = a_r[0, 0]
        cols = jax.lax.broadcasted_iota(jnp.int32, (BLK_E, 64), 1)
        oh = (cols == a[:, None]).astype(jnp.float32)
        o_r[...] = jnp.dot(oh, tab_r[...], preferred_element_type=jnp.float32)

    return pl.pallas_call(
        body,
        grid=(NBLK_E,),
        in_specs=[
            pl.BlockSpec((1, 1, BLK_E), lambda i: (i, 0, 0)),
            pl.BlockSpec((64, NFL), lambda i: (0, 0)),
        ],
        out_specs=pl.BlockSpec((BLK_E, NFL), lambda i: (i, 0)),
        out_shape=jax.ShapeDtypeStruct((E, NFL), jnp.float32),
    )(attr3, bond_tab)


def _edge_t(anbr_r, ainit_r, ea_r, W_r, b_r):
    return (
        jnp.dot(anbr_r[:, 0:AFL], W_r[0:AFL, :],
                preferred_element_type=jnp.float32)
        + jnp.dot(ainit_r[:, 0:AFL], W_r[AFL:2 * AFL, :],
                  preferred_element_type=jnp.float32)
        + jnp.dot(ea_r[...], W_r[2 * AFL:, :],
                  preferred_element_type=jnp.float32)
        + b_r[...]
    )


def _conv_stats(gath, ew, Wcat, bcat):
    def body(anbr_r, ainit_r, ea_r, W_r, b_r, acc_r):
        @pl.when(pl.program_id(0) == 0)
        def _init():
            acc_r[...] = jnp.zeros((2, 2 * AFL + NFL), jnp.float32)

        t = _edge_t(anbr_r, ainit_r, ea_r, W_r, b_r)
        s = jnp.sum(t, axis=0)
        ss = jnp.sum(t * t, axis=0)
        acc_r[...] += jnp.concatenate([s[None, :], ss[None, :]], axis=0)

    return pl.pallas_call(
        body,
        grid=(NBLK_E,),
        in_specs=[
            pl.BlockSpec((BLK_E, _GW), lambda i: (i, 0)),
            pl.BlockSpec((BLK_E, _GW), lambda i: (i + NBLK_E, 0)),
            pl.BlockSpec((BLK_E, NFL), lambda i: (i, 0)),
            pl.BlockSpec((2 * AFL + NFL, 2 * AFL + NFL), lambda i: (0, 0)),
            pl.BlockSpec((1, 2 * AFL + NFL), lambda i: (0, 0)),
        ],
        out_specs=pl.BlockSpec((2, 2 * AFL + NFL), lambda i: (0, 0)),
        out_shape=jax.ShapeDtypeStruct((2, 2 * AFL + NFL), jnp.float32),
    )(gath, gath, ew, Wcat, bcat)


def _conv_apply(gath, ew, Wcat, bcat, acc, gcat, betacat):
    def body(anbr_r, ainit_r, ea_r, W_r, b_r, acc_r, g_r, bt_r, nbr_r, ewo_r):
        t = _edge_t(anbr_r, ainit_r, ea_r, W_r, b_r)
        m = acc_r[0, :] * (1.0 / E)
        var = acc_r[1, :] * (1.0 / E) - m * m
        rstd = jax.lax.rsqrt(var + 1e-5)
        tn = (t - m[None, :]) * (rstd * g_r[0, :])[None, :] + bt_r[0, :][None, :]
        filt = _sigmoid(tn[:, 0:AFL])
        core = _softplus(tn[:, AFL:2 * AFL])
        prod = filt * core
        nbr_r[...] = jnp.sum(
            prod.reshape(BLK_E // NUM_NBR, NUM_NBR, AFL), axis=1)
        ewo_r[...] = _softplus(ea_r[...] + tn[:, 2 * AFL:])

    return pl.pallas_call(
        body,
        grid=(NBLK_E,),
        in_specs=[
            pl.BlockSpec((BLK_E, _GW), lambda i: (i, 0)),
            pl.BlockSpec((BLK_E, _GW), lambda i: (i + NBLK_E, 0)),
            pl.BlockSpec((BLK_E, NFL), lambda i: (i, 0)),
            pl.BlockSpec((2 * AFL + NFL, 2 * AFL + NFL), lambda i: (0, 0)),
            pl.BlockSpec((1, 2 * AFL + NFL), lambda i: (0, 0)),
            pl.BlockSpec((2, 2 * AFL + NFL), lambda i: (0, 0)),
            pl.BlockSpec((1, 2 * AFL + NFL), lambda i: (0, 0)),
            pl.BlockSpec((1, 2 * AFL + NFL), lambda i: (0, 0)),
        ],
        out_specs=(
            pl.BlockSpec((BLK_E // NUM_NBR, AFL), lambda i: (i, 0)),
            pl.BlockSpec((BLK_E, NFL), lambda i: (i, 0)),
        ),
        out_shape=(
            jax.ShapeDtypeStruct((N, AFL), jnp.float32),
            jax.ShapeDtypeStruct((E, NFL), jnp.float32),
        ),
    )(gath, gath, ew, Wcat, bcat, acc, gcat, betacat)


def _node_update(h, nbr, g4, b4):
    def body(h_r, n_r, g_r, b_r, o_r):
        nb = n_r[...]
        m = jnp.sum(nb, axis=0) * (1.0 / N)
        d = nb - m[None, :]
        var = jnp.sum(d * d, axis=0) * (1.0 / N)
        rstd = jax.lax.rsqrt(var + 1e-5)
        o_r[...] = _softplus(h_r[...] + d * (rstd * g_r[0, :])[None, :]
                             + b_r[0, :][None, :])

    return pl.pallas_call(
        body,
        in_specs=[
            pl.BlockSpec((N, AFL), lambda: (0, 0)),
            pl.BlockSpec((N, AFL), lambda: (0, 0)),
            pl.BlockSpec((1, AFL), lambda: (0, 0)),
            pl.BlockSpec((1, AFL), lambda: (0, 0)),
        ],
        out_specs=pl.BlockSpec((N, AFL), lambda: (0, 0)),
        out_shape=jax.ShapeDtypeStruct((N, AFL), jnp.float32),
    )(h, nbr, g4, b4)


def _head_pool(h0, h1, h2, h3, batch3, Wfc, bfc, l1W, l1b):
    def body(h0_r, h1_r, h2_r, h3_r, b_r, Wfc_r, bfc_r, l1W_r, l1b_r,
             s_r, c_r):
        @pl.when(pl.program_id(0) == 0)
        def _init():
            s_r[...] = jnp.zeros((NGRAPH, H), jnp.float32)
            c_r[...] = jnp.zeros((1, NGRAPH), jnp.float32)

        z = (
            jnp.dot(h0_r[...], Wfc_r[0:AFL, :],
                    preferred_element_type=jnp.float32)
            + jnp.dot(h1_r[...], Wfc_r[AFL:2 * AFL, :],
                      preferred_element_type=jnp.float32)
            + jnp.dot(h2_r[...], Wfc_r[2 * AFL:3 * AFL, :],
                      preferred_element_type=jnp.float32)
            + jnp.dot(h3_r[...], Wfc_r[3 * AFL:, :],
                      preferred_element_type=jnp.float32)
            + bfc_r[...]
        )
        z = _softplus(jnp.dot(z, l1W_r[...],
                              preferred_element_type=jnp.float32) + l1b_r[...])
        b = b_r[0, 0]
        rows = jax.lax.broadcasted_iota(jnp.int32, (NGRAPH, BLK_N), 0)
        ohT = (rows == b[None, :]).astype(jnp.float32)
        s_r[...] += jnp.dot(ohT, z, preferred_element_type=jnp.float32)
        c_r[...] += jnp.sum(ohT, axis=1)[None, :]

    return pl.pallas_call(
        body,
        grid=(NBLK_N,),
        in_specs=[
            pl.BlockSpec((BLK_N, AFL), lambda i: (i, 0)),
            pl.BlockSpec((BLK_N, AFL), lambda i: (i, 0)),
            pl.BlockSpec((BLK_N, AFL), lambda i: (i, 0)),
            pl.BlockSpec((BLK_N, AFL), lambda i: (i, 0)),
            pl.BlockSpec((1, 1, BLK_N), lambda i: (i, 0, 0)),
            pl.BlockSpec((4 * AFL, H), lambda i: (0, 0)),
            pl.BlockSpec((1, H), lambda i: (0, 0)),
            pl.BlockSpec((H, H), lambda i: (0, 0)),
            pl.BlockSpec((1, H), lambda i: (0, 0)),
        ],
        out_specs=(
            pl.BlockSpec((NGRAPH, H), lambda i: (0, 0)),
            pl.BlockSpec((1, NGRAPH), lambda i: (0, 0)),
        ),
        out_shape=(
            jax.ShapeDtypeStruct((NGRAPH, H), jnp.float32),
            jax.ShapeDtypeStruct((1, NGRAPH), jnp.float32),
        ),
    )(h0, h1, h2, h3, batch3, Wfc, bfc, l1W, l1b)


def _head_out(s, cnt, l2W, l2b, WoutP, boutP):
    def body(s_r, c_r, l2W_r, l2b_r, Wo_r, bo_r, o_r):
        c = jnp.maximum(c_r[0, :], 1.0)
        mean = s_r[...] * (1.0 / c)[:, None]
        z = _softplus(jnp.dot(mean, l2W_r[...],
                              preferred_element_type=jnp.float32) + l2b_r[...])
        o_r[...] = jnp.dot(z, Wo_r[...],
                           preferred_element_type=jnp.float32) + bo_r[...]

    return pl.pallas_call(
        body,
        in_specs=[
            pl.BlockSpec((NGRAPH, H), lambda: (0, 0)),
            pl.BlockSpec((1, NGRAPH), lambda: (0, 0)),
            pl.BlockSpec((H, H), lambda: (0, 0)),
            pl.BlockSpec((1, H), lambda: (0, 0)),
            pl.BlockSpec((H, H), lambda: (0, 0)),
            pl.BlockSpec((1, H), lambda: (0, 0)),
        ],
        out_specs=pl.BlockSpec((NGRAPH, H), lambda: (0, 0)),
        out_shape=jax.ShapeDtypeStruct((NGRAPH, H), jnp.float32),
    )(s, cnt, l2W, l2b, WoutP, boutP)


def kernel(x, edge_index, edge_attr, batch, params):
    x = x.astype(jnp.int32)
    ei = edge_index.astype(jnp.int32)
    attr = edge_attr.astype(jnp.int32)
    batch = batch.astype(jnp.int32)

    idx_all = jnp.concatenate([ei[1], ei[0]])

    atom_tab = jnp.concatenate(params['atom_emb'], axis=0)
    atom_tab = jnp.pad(atom_tab, ((0, _ATOM_TOT_PAD - atom_tab.shape[0]),
                                  (0, 0)))
    bond_tab = jnp.pad(params['bond_emb'], ((0, 64 - 51), (0, 0)))

    h = _atom_embed(x.reshape(NBLK_N, BLK_N, 9), atom_tab)
    ew = _bond_embed(attr.reshape(NBLK_E, 1, BLK_E), bond_tab)

    hs = [h]
    for p in params['convs']:
        Wcat = jnp.concatenate([p['Wc'], p['Wf'], p['Wb']], axis=1)
        bcat = jnp.concatenate([p['bc'], p['bf'], p['bb']]).reshape(1, -1)
        gcat = jnp.concatenate([p['g1'], p['g2'], p['g3']]).reshape(1, -1)
        btcat = jnp.concatenate([p['b1'], p['b2'], p['b3']]).reshape(1, -1)

        gath = _gather_edges(jnp.pad(h, ((0, 0), (0, _GW - AFL))), idx_all)
        acc = _conv_stats(gath, ew, Wcat, bcat)
        nbr, ew = _conv_apply(gath, ew, Wcat, bcat, acc, gcat, btcat)
        h = _node_update(h, nbr, p['g4'].reshape(1, AFL),
                         p['b4'].reshape(1, AFL))
        hs.append(h)

    s, cnt = _head_pool(hs[0], hs[1], hs[2], hs[3],
                        batch.reshape(NBLK_N, 1, BLK_N),
                        params['W_fc'], params['b_fc'].reshape(1, H),
                        params['l1_W'], params['l1_b'].reshape(1, H))
    WoutP = jnp.pad(params['Wout'], ((0, 0), (0, H - 1)))
    boutP = jnp.broadcast_to(params['bout'].reshape(1, 1), (1, H))
    o = _head_out(s, cnt, params['l2_W'], params['l2_b'].reshape(1, H),
                  WoutP, boutP)
    return o[:, 0:1]


# speedup vs baseline: 3.3237x; 1.0372x over previous
import functools

import jax
import jax.numpy as jnp
from jax import lax
from jax.experimental import pallas as pl
from jax.experimental.pallas import tpu as pltpu
from jax.experimental.pallas import tpu_sc as plsc

N = 10000
E = 160000
NUM_NBR = 16
AFL = 64
NFL = 128
H = 128
NGRAPH = 64

BLK_E = 3200
NBLK_E = E // BLK_E
BLK_N = 2000
NBLK_N = N // BLK_N

TWO_E = 2 * E
_NW = 32
_BPW = TWO_E // _NW
_CH = 80
_NBUF = 5
_NGRP = _BPW // (_CH * _NBUF)

_ATOM_DIMS = [100, 18, 7, 12, 10, 10, 10, 10, 10]
_ATOM_OFF = [0, 100, 118, 125, 137, 147, 157, 167, 177]
_ATOM_TOT_PAD = 192


def _softplus(x):
    return jnp.maximum(x, 0.0) + jnp.log(1.0 + jnp.exp(-jnp.abs(x)))


def _sigmoid(x):
    return 1.0 / (1.0 + jnp.exp(-x))


_GW = 128


def _gather_edges(h, idx_all):
    mesh = plsc.VectorSubcoreMesh(core_axis_name="c", subcore_axis_name="s")

    @functools.partial(
        pl.kernel,
        mesh=mesh,
        out_type=jax.ShapeDtypeStruct((TWO_E, _GW), jnp.float32),
        scratch_types=(
            [pltpu.VMEM((_BPW,), jnp.int32)]
            + [pltpu.VMEM((_CH, _GW), jnp.float32) for _ in range(_NBUF)]
            + [pltpu.SemaphoreType.DMA for _ in range(2 * _NBUF)]
        ),
    )
    def k(h_hbm, idx_hbm, out_hbm, idx_v, *rest):
        bufs = rest[:_NBUF]
        gsems = rest[_NBUF:2 * _NBUF]
        wsems = rest[2 * _NBUF:]
        wid = lax.axis_index("s") * 2 + lax.axis_index("c")
        base = wid * _BPW
        pltpu.sync_copy(idx_hbm.at[pl.ds(base, _BPW)], idx_v)

        def group(j, carry):
            gd = []
            for b in range(_NBUF):
                off = pl.multiple_of((j * _NBUF + b) * _CH, 8)
                gd.append(pltpu.async_copy(
                    h_hbm.at[idx_v.at[pl.ds(off, _CH)]], bufs[b], gsems[b]))
            wd = []
            for b in range(_NBUF):
                off = pl.multiple_of((j * _NBUF + b) * _CH, 8)
                gd[b].wait()
                wd.append(pltpu.async_copy(
                    bufs[b], out_hbm.at[pl.ds(base + off, _CH)], wsems[b]))
            for d in wd:
                d.wait()
            return carry

        lax.fori_loop(0, _NGRP, group, 0)

    return k(h, idx_all)


def _atom_embed(x3, atom_tab):
    def body(x_r, tab_r, o_r):
        xb = x_r[0]
        mh = jnp.zeros((BLK_N, _ATOM_TOT_PAD), jnp.float32)
        cols = jax.lax.broadcasted_iota(jnp.int32, (BLK_N, _ATOM_TOT_PAD), 1)
        for i in range(9):
            tgt = xb[:, i] + _ATOM_OFF[i]
            mh = mh + (cols == tgt[:, None]).astype(jnp.float32)
        o_r[...] = jnp.dot(mh, tab_r[...], preferred_element_type=jnp.float32)

    return pl.pallas_call(
        body,
        grid=(NBLK_N,),
        in_specs=[
            pl.BlockSpec((1, BLK_N, 9), lambda i: (i, 0, 0)),
            pl.BlockSpec((_ATOM_TOT_PAD, AFL), lambda i: (0, 0)),
        ],
        out_specs=pl.BlockSpec((BLK_N, AFL), lambda i: (i, 0)),
        out_shape=jax.ShapeDtypeStruct((N, AFL), jnp.float32),
    )(x3, atom_tab)


def _bond_embed(attr3, bond_tab):
    def body(a_r, tab_r, o_r):
        a = a_r[0, 0]
        cols = jax.lax.broadcasted_iota(jnp.int32, (BLK_E, 64), 1)
        oh = (cols == a[:, None]).astype(jnp.float32)
        o_r[...] = jnp.dot(oh, tab_r[...], preferred_element_type=jnp.float32)

    return pl.pallas_call(
        body,
        grid=(NBLK_E,),
        in_specs=[
            pl.BlockSpec((1, 1, BLK_E), lambda i: (i, 0, 0)),
            pl.BlockSpec((64, NFL), lambda i: (0, 0)),
        ],
        out_specs=pl.BlockSpec((BLK_E, NFL), lambda i: (i, 0)),
        out_shape=jax.ShapeDtypeStruct((E, NFL), jnp.float32),
    )(attr3, bond_tab)


def _edge_t(anbr_r, ainit_r, ea_r, W_r, b_r):
    return (
        jnp.dot(anbr_r[:, 0:AFL], W_r[0:AFL, :],
                preferred_element_type=jnp.float32)
        + jnp.dot(ainit_r[:, 0:AFL], W_r[AFL:2 * AFL, :],
                  preferred_element_type=jnp.float32)
        + jnp.dot(ea_r[...], W_r[2 * AFL:, :],
                  preferred_element_type=jnp.float32)
        + b_r[...]
    )


def _conv_stats(gath, ew, Wcat, bcat):
    def body(anbr_r, ainit_r, ea_r, W_r, b_r, acc_r, t_r):
        @pl.when(pl.program_id(0) == 0)
        def _init():
            acc_r[...] = jnp.zeros((2, 2 * AFL + NFL), jnp.float32)

        t = _edge_t(anbr_r, ainit_r, ea_r, W_r, b_r)
        s = jnp.sum(t, axis=0)
        ss = jnp.sum(t * t, axis=0)
        acc_r[...] += jnp.concatenate([s[None, :], ss[None, :]], axis=0)
        t_r[...] = t.astype(jnp.bfloat16)

    return pl.pallas_call(
        body,
        grid=(NBLK_E,),
        in_specs=[
            pl.BlockSpec((BLK_E, _GW), lambda i: (i, 0)),
            pl.BlockSpec((BLK_E, _GW), lambda i: (i + NBLK_E, 0)),
            pl.BlockSpec((BLK_E, NFL), lambda i: (i, 0)),
            pl.BlockSpec((2 * AFL + NFL, 2 * AFL + NFL), lambda i: (0, 0)),
            pl.BlockSpec((1, 2 * AFL + NFL), lambda i: (0, 0)),
        ],
        out_specs=(
            pl.BlockSpec((2, 2 * AFL + NFL), lambda i: (0, 0)),
            pl.BlockSpec((BLK_E, 2 * AFL + NFL), lambda i: (i, 0)),
        ),
        out_shape=(
            jax.ShapeDtypeStruct((2, 2 * AFL + NFL), jnp.float32),
            jax.ShapeDtypeStruct((E, 2 * AFL + NFL), jnp.bfloat16),
        ),
    )(gath, gath, ew, Wcat, bcat)


def _conv_apply(tbf, ew, acc, gcat, betacat):
    def body(t_r, ea_r, acc_r, g_r, bt_r, nbr_r, ewo_r):
        t = t_r[...].astype(jnp.float32)
        m = acc_r[0, :] * (1.0 / E)
        var = acc_r[1, :] * (1.0 / E) - m * m
        rstd = jax.lax.rsqrt(var + 1e-5)
        tn = (t - m[None, :]) * (rstd * g_r[0, :])[None, :] + bt_r[0, :][None, :]
        filt = _sigmoid(tn[:, 0:AFL])
        core = _softplus(tn[:, AFL:2 * AFL])
        prod = filt * core
        nbr_r[...] = jnp.sum(
            prod.reshape(BLK_E // NUM_NBR, NUM_NBR, AFL), axis=1)
        ewo_r[...] = _softplus(ea_r[...] + tn[:, 2 * AFL:])

    return pl.pallas_call(
        body,
        grid=(NBLK_E,),
        in_specs=[
            pl.BlockSpec((BLK_E, 2 * AFL + NFL), lambda i: (i, 0)),
            pl.BlockSpec((BLK_E, NFL), lambda i: (i, 0)),
            pl.BlockSpec((2, 2 * AFL + NFL), lambda i: (0, 0)),
            pl.BlockSpec((1, 2 * AFL + NFL), lambda i: (0, 0)),
            pl.BlockSpec((1, 2 * AFL + NFL), lambda i: (0, 0)),
        ],
        out_specs=(
            pl.BlockSpec((BLK_E // NUM_NBR, AFL), lambda i: (i, 0)),
            pl.BlockSpec((BLK_E, NFL), lambda i: (i, 0)),
        ),
        out_shape=(
            jax.ShapeDtypeStruct((N, AFL), jnp.float32),
            jax.ShapeDtypeStruct((E, NFL), jnp.float32),
        ),
    )(tbf, ew, acc, gcat, betacat)


def _node_update(h, nbr, g4, b4):
    def body(h_r, n_r, g_r, b_r, o_r):
        nb = n_r[...]
        m = jnp.sum(nb, axis=0) * (1.0 / N)
        d = nb - m[None, :]
        var = jnp.sum(d * d, axis=0) * (1.0 / N)
        rstd = jax.lax.rsqrt(var + 1e-5)
        o_r[...] = _softplus(h_r[...] + d * (rstd * g_r[0, :])[None, :]
                             + b_r[0, :][None, :])

    return pl.pallas_call(
        body,
        in_specs=[
            pl.BlockSpec((N, AFL), lambda: (0, 0)),
            pl.BlockSpec((N, AFL), lambda: (0, 0)),
            pl.BlockSpec((1, AFL), lambda: (0, 0)),
            pl.BlockSpec((1, AFL), lambda: (0, 0)),
        ],
        out_specs=pl.BlockSpec((N, AFL), lambda: (0, 0)),
        out_shape=jax.ShapeDtypeStruct((N, AFL), jnp.float32),
    )(h, nbr, g4, b4)


def _head_pool(h0, h1, h2, h3, batch3, Wfc, bfc, l1W, l1b):
    def body(h0_r, h1_r, h2_r, h3_r, b_r, Wfc_r, bfc_r, l1W_r, l1b_r,
             s_r, c_r):
        @pl.when(pl.program_id(0) == 0)
        def _init():
            s_r[...] = jnp.zeros((NGRAPH, H), jnp.float32)
            c_r[...] = jnp.zeros((1, NGRAPH), jnp.float32)

        z = (
            jnp.dot(h0_r[...], Wfc_r[0:AFL, :],
                    preferred_element_type=jnp.float32)
            + jnp.dot(h1_r[...], Wfc_r[AFL:2 * AFL, :],
                      preferred_element_type=jnp.float32)
            + jnp.dot(h2_r[...], Wfc_r[2 * AFL:3 * AFL, :],
                      preferred_element_type=jnp.float32)
            + jnp.dot(h3_r[...], Wfc_r[3 * AFL:, :],
                      preferred_element_type=jnp.float32)
            + bfc_r[...]
        )
        z = _softplus(jnp.dot(z, l1W_r[...],
                              preferred_element_type=jnp.float32) + l1b_r[...])
        b = b_r[0, 0]
        rows = jax.lax.broadcasted_iota(jnp.int32, (NGRAPH, BLK_N), 0)
        ohT = (rows == b[None, :]).astype(jnp.float32)
        s_r[...] += jnp.dot(ohT, z, preferred_element_type=jnp.float32)
        c_r[...] += jnp.sum(ohT, axis=1)[None, :]

    return pl.pallas_call(
        body,
        grid=(NBLK_N,),
        in_specs=[
            pl.BlockSpec((BLK_N, AFL), lambda i: (i, 0)),
            pl.BlockSpec((BLK_N, AFL), lambda i: (i, 0)),
            pl.BlockSpec((BLK_N, AFL), lambda i: (i, 0)),
            pl.BlockSpec((BLK_N, AFL), lambda i: (i, 0)),
            pl.BlockSpec((1, 1, BLK_N), lambda i: (i, 0, 0)),
            pl.BlockSpec((4 * AFL, H), lambda i: (0, 0)),
            pl.BlockSpec((1, H), lambda i: (0, 0)),
            pl.BlockSpec((H, H), lambda i: (0, 0)),
            pl.BlockSpec((1, H), lambda i: (0, 0)),
        ],
        out_specs=(
            pl.BlockSpec((NGRAPH, H), lambda i: (0, 0)),
            pl.BlockSpec((1, NGRAPH), lambda i: (0, 0)),
        ),
        out_shape=(
            jax.ShapeDtypeStruct((NGRAPH, H), jnp.float32),
            jax.ShapeDtypeStruct((1, NGRAPH), jnp.float32),
        ),
    )(h0, h1, h2, h3, batch3, Wfc, bfc, l1W, l1b)


def _head_out(s, cnt, l2W, l2b, WoutP, boutP):
    def body(s_r, c_r, l2W_r, l2b_r, Wo_r, bo_r, o_r):
        c = jnp.maximum(c_r[0, :], 1.0)
        mean = s_r[...] * (1.0 / c)[:, None]
        z = _softplus(jnp.dot(mean, l2W_r[...],
                              preferred_element_type=jnp.float32) + l2b_r[...])
        o_r[...] = jnp.dot(z, Wo_r[...],
                           preferred_element_type=jnp.float32) + bo_r[...]

    return pl.pallas_call(
        body,
        in_specs=[
            pl.BlockSpec((NGRAPH, H), lambda: (0, 0)),
            pl.BlockSpec((1, NGRAPH), lambda: (0, 0)),
            pl.BlockSpec((H, H), lambda: (0, 0)),
            pl.BlockSpec((1, H), lambda: (0, 0)),
            pl.BlockSpec((H, H), lambda: (0, 0)),
            pl.BlockSpec((1, H), lambda: (0, 0)),
        ],
        out_specs=pl.BlockSpec((NGRAPH, H), lambda: (0, 0)),
        out_shape=jax.ShapeDtypeStruct((NGRAPH, H), jnp.float32),
    )(s, cnt, l2W, l2b, WoutP, boutP)


def kernel(x, edge_index, edge_attr, batch, params):
    x = x.astype(jnp.int32)
    ei = edge_index.astype(jnp.int32)
    attr = edge_attr.astype(jnp.int32)
    batch = batch.astype(jnp.int32)

    idx_all = jnp.concatenate([ei[1], ei[0]])

    atom_tab = jnp.concatenate(params['atom_emb'], axis=0)
    atom_tab = jnp.pad(atom_tab, ((0, _ATOM_TOT_PAD - atom_tab.shape[0]),
                                  (0, 0)))
    bond_tab = jnp.pad(params['bond_emb'], ((0, 64 - 51), (0, 0)))

    h = _atom_embed(x.reshape(NBLK_N, BLK_N, 9), atom_tab)
    ew = _bond_embed(attr.reshape(NBLK_E, 1, BLK_E), bond_tab)

    hs = [h]
    for p in params['convs']:
        Wcat = jnp.concatenate([p['Wc'], p['Wf'], p['Wb']], axis=1)
        bcat = jnp.concatenate([p['bc'], p['bf'], p['bb']]).reshape(1, -1)
        gcat = jnp.concatenate([p['g1'], p['g2'], p['g3']]).reshape(1, -1)
        btcat = jnp.concatenate([p['b1'], p['b2'], p['b3']]).reshape(1, -1)

        gath = _gather_edges(jnp.pad(h, ((0, 0), (0, _GW - AFL))), idx_all)
        acc, tbf = _conv_stats(gath, ew, Wcat, bcat)
        nbr, ew = _conv_apply(tbf, ew, acc, gcat, btcat)
        h = _node_update(h, nbr, p['g4'].reshape(1, AFL),
                         p['b4'].reshape(1, AFL))
        hs.append(h)

    s, cnt = _head_pool(hs[0], hs[1], hs[2], hs[3],
                        batch.reshape(NBLK_N, 1, BLK_N),
                        params['W_fc'], params['b_fc'].reshape(1, H),
                        params['l1_W'], params['l1_b'].reshape(1, H))
    WoutP = jnp.pad(params['Wout'], ((0, 0), (0, H - 1)))
    boutP = jnp.broadcast_to(params['bout'].reshape(1, 1), (1, H))
    o = _head_out(s, cnt, params['l2_W'], params['l2_b'].reshape(1, H),
                  WoutP, boutP)
    return o[:, 0:1]
